# Initial kernel scaffold; baseline (speedup 1.0000x reference)
import functools

import jax
import jax.numpy as jnp
from jax import lax
from jax.experimental import pallas as pl
from jax.experimental.pallas import tpu as pltpu
from jax.experimental.pallas import tpu_sc as plsc

_N = 10000
_E = 320000
_D = 128
_NC = 2
_NS = 16
_CH = 80
_EPT = _E // _NS
_NCHUNK = _EPT // _CH
_RPT = _N // _NS


def _sc_segsum_body(ei_ref, ea_ref, x_ref, znd_ref, zn16_ref,
                    bd_ref, bs_ref, as_ref, at_ref, cd_ref, cs_ref,
                    acc, cnt, eabuf, xbuf, idi, idj, ones):
    c = lax.axis_index("c")
    s = lax.axis_index("s")
    r0 = s * _RPT
    ebase = s * _EPT

    def _ones_body(k, carry):
        ones[k, :] = jnp.ones((16,), jnp.float32)
        return carry
    lax.fori_loop(0, _CH, _ones_body, 0)

    pltpu.sync_copy(znd_ref.at[pl.ds(r0, _RPT)], acc.at[pl.ds(r0, _RPT)])
    plsc.subcore_barrier()

    key_off = (1 - c) * _E

    def _phase_a(k, carry):
        e0 = ebase + k * _CH
        pltpu.sync_copy(ei_ref.at[pl.ds(key_off + e0, _CH)], idi)
        pltpu.sync_copy(ea_ref.at[pl.ds(e0, _CH)], eabuf)
        pltpu.sync_copy(eabuf, acc.at[idi], add=True)
        return carry
    lax.fori_loop(0, _NCHUNK, _phase_a, 0)
    plsc.subcore_barrier()

    @pl.when(c == 0)
    def _():
        pltpu.sync_copy(acc.at[pl.ds(r0, _RPT)], bd_ref.at[pl.ds(r0, _RPT)])

    @pl.when(c == 1)
    def _():
        pltpu.sync_copy(acc.at[pl.ds(r0, _RPT)], bs_ref.at[pl.ds(r0, _RPT)])
    plsc.subcore_barrier()

    pltpu.sync_copy(znd_ref.at[pl.ds(r0, _RPT)], acc.at[pl.ds(r0, _RPT)])
    pltpu.sync_copy(zn16_ref.at[pl.ds(r0, _RPT)], cnt.at[pl.ds(r0, _RPT)])
    plsc.subcore_barrier()

    g_off = c * _E
    s_off = (1 - c) * _E

    def _phase_b(k, carry):
        e0 = ebase + k * _CH
        pltpu.sync_copy(ei_ref.at[pl.ds(g_off + e0, _CH)], idj)
        pltpu.sync_copy(ei_ref.at[pl.ds(s_off + e0, _CH)], idi)
        pltpu.sync_copy(x_ref.at[idj], xbuf)
        pltpu.sync_copy(xbuf, acc.at[idi], add=True)
        pltpu.sync_copy(ones, cnt.at[idi], add=True)
        return carry
    lax.fori_loop(0, _NCHUNK, _phase_b, 0)
    plsc.subcore_barrier()

    @pl.when(c == 0)
    def _():
        pltpu.sync_copy(acc.at[pl.ds(r0, _RPT)], as_ref.at[pl.ds(r0, _RPT)])
        pltpu.sync_copy(cnt.at[pl.ds(r0, _RPT)], cd_ref.at[pl.ds(r0, _RPT)])

    @pl.when(c == 1)
    def _():
        pltpu.sync_copy(acc.at[pl.ds(r0, _RPT)], at_ref.at[pl.ds(r0, _RPT)])
        pltpu.sync_copy(cnt.at[pl.ds(r0, _RPT)], cs_ref.at[pl.ds(r0, _RPT)])


_sc_segsums = pl.kernel(
    _sc_segsum_body,
    out_type=(
        jax.ShapeDtypeStruct((_N, _D), jnp.float32),
        jax.ShapeDtypeStruct((_N, _D), jnp.float32),
        jax.ShapeDtypeStruct((_N, _D), jnp.float32),
        jax.ShapeDtypeStruct((_N, _D), jnp.float32),
        jax.ShapeDtypeStruct((_N, 16), jnp.float32),
        jax.ShapeDtypeStruct((_N, 16), jnp.float32),
    ),
    mesh=plsc.VectorSubcoreMesh(core_axis_name="c", subcore_axis_name="s",
                                num_cores=_NC, num_subcores=_NS),
    scratch_types=(
        pltpu.VMEM_SHARED((_N, _D), jnp.float32),
        pltpu.VMEM_SHARED((_N, 16), jnp.float32),
        pltpu.VMEM((_CH, _D), jnp.float32),
        pltpu.VMEM((_CH, _D), jnp.float32),
        pltpu.VMEM((_CH,), jnp.int32),
        pltpu.VMEM((_CH,), jnp.int32),
        pltpu.VMEM((_CH, 16), jnp.float32),
    ),
)


def _tc_combine_body(x_ref, as_ref, at_ref, bd_ref, bs_ref, cd_ref, cs_ref,
                     w_ref, b_ref, o_ref):
    f32 = jnp.float32
    x = x_ref[...]
    o = jnp.dot(as_ref[...], w_ref[0], preferred_element_type=f32)
    o += jnp.dot(at_ref[...], w_ref[1], preferred_element_type=f32)
    o += jnp.dot(bd_ref[...], w_ref[2], preferred_element_type=f32)
    o += jnp.dot(bs_ref[...], w_ref[3], preferred_element_type=f32)
    xqs = jnp.dot(x, w_ref[4], preferred_element_type=f32) + b_ref[0]
    xqt = jnp.dot(x, w_ref[5], preferred_element_type=f32) + b_ref[1]
    o += cd_ref[:, 0:1] * xqs + cs_ref[:, 0:1] * xqt
    o += jnp.dot(x, w_ref[6], preferred_element_type=f32)
    o_ref[...] = o


def _tc_combine(x, As, At, Bd, Bs, cd, cs, Wstack, bstack):
    blk = 2000
    grid = _N // blk
    row = lambda i: (i, 0)
    return pl.pallas_call(
        _tc_combine_body,
        grid=(grid,),
        in_specs=[
            pl.BlockSpec((blk, _D), row),
            pl.BlockSpec((blk, _D), row),
            pl.BlockSpec((blk, _D), row),
            pl.BlockSpec((blk, _D), row),
            pl.BlockSpec((blk, _D), row),
            pl.BlockSpec((blk, 16), row),
            pl.BlockSpec((blk, 16), row),
            pl.BlockSpec((7, _D, _D), lambda i: (0, 0, 0)),
            pl.BlockSpec((2, _D), lambda i: (0, 0)),
        ],
        out_specs=pl.BlockSpec((blk, _D), row),
        out_shape=jax.ShapeDtypeStruct((_N, _D), jnp.float32),
    )(x, As, At, Bd, Bs, cd, cs, Wstack, bstack)


def kernel(x, edge_index, edge_attr, multimodal_features,
           Wn_s2t, We_s2t, Wm_s2t, bm_s2t,
           Wn_t2s, We_t2s, Wm_t2s, bm_t2s):
    D = x.shape[1]
    ei = edge_index.reshape(2 * _E)
    znd = jnp.zeros((_N, _D), jnp.float32)
    zn16 = jnp.zeros((_N, 16), jnp.float32)

    Bd, Bs, As, At, cd, cs = _sc_segsums(ei, edge_attr, x, znd, zn16)

    def mats(Wn, We, Wm):
        W1, W2, W3 = Wm[:, :D], Wm[:, D:2 * D], Wm[:, 2 * D:]
        return Wn.T @ W1.T, Wn.T @ W2.T, We.T @ W3.T

    Ps, Qs, Rs = mats(Wn_s2t, We_s2t, Wm_s2t)
    Pt, Qt, Rt = mats(Wn_t2s, We_t2s, Wm_t2s)
    Wstack = 0.5 * jnp.stack([Ps, Pt, Rs, Rt, Qs, Qt, (Wn_s2t + Wn_t2s).T])
    bstack = 0.5 * jnp.stack([bm_s2t, bm_t2s])

    node_out = _tc_combine(x, As, At, Bd, Bs, cd, cs, Wstack, bstack)
    return (node_out, edge_attr)


# speedup vs baseline: 3.5083x; 3.5083x over previous
import jax
import jax.numpy as jnp
from jax import lax
from jax.experimental import pallas as pl
from jax.experimental.pallas import tpu as pltpu
from jax.experimental.pallas import tpu_sc as plsc

_N = 10000
_E = 320000
_D = 128
_NC = 2
_NS = 16
_CH = 80
_EPT = _E // _NS
_NCHUNK = _EPT // _CH
_NP = 10240
_RPT = _NP // _NS
_QZ = _RPT // _CH

_MESH = plsc.VectorSubcoreMesh(core_axis_name="c", subcore_axis_name="s",
                               num_cores=_NC, num_subcores=_NS)


def _zero_acc(zsrc_ref, buf, acc, r0):
    pltpu.sync_copy(zsrc_ref, buf)

    def _z(q, carry):
        pltpu.sync_copy(buf, acc.at[pl.ds(r0 + q * _CH, _CH)])
        return carry
    lax.fori_loop(0, _QZ, _z, 0)


def _drain_acc(acc, buf, out_ref, r0):
    def _d(q, carry):
        pltpu.sync_copy(acc.at[pl.ds(r0 + q * _CH, _CH)], buf)
        pltpu.sync_copy(buf, out_ref.at[pl.ds(r0 + q * _CH, _CH)])
        return carry
    lax.fori_loop(0, _QZ, _d, 0)


def _ea_body(ei_ref, ea_ref, znd_ref, bd_ref, bs_ref, acc, eabuf, idi):
    c = lax.axis_index("c")
    s = lax.axis_index("s")
    r0 = s * _RPT
    ebase = s * _EPT
    _zero_acc(znd_ref, eabuf, acc, r0)
    plsc.subcore_barrier()

    key_off = (1 - c) * _E

    def _loop(k, carry):
        e0 = ebase + k * _CH
        pltpu.sync_copy(ei_ref.at[pl.ds(key_off + e0, _CH)], idi)
        pltpu.sync_copy(ea_ref.at[pl.ds(e0, _CH)], eabuf)
        pltpu.sync_copy(eabuf, acc.at[idi], add=True)
        return carry
    lax.fori_loop(0, _NCHUNK, _loop, 0)
    plsc.subcore_barrier()

    @pl.when(c == 0)
    def _():
        _drain_acc(acc, eabuf, bd_ref, r0)

    @pl.when(c == 1)
    def _():
        _drain_acc(acc, eabuf, bs_ref, r0)


def _x_body(ei_ref, x_ref, znd_ref, as_ref, at_ref, acc, xbuf, idi, idj, sem):
    c = lax.axis_index("c")
    s = lax.axis_index("s")
    r0 = s * _RPT
    ebase = s * _EPT
    _zero_acc(znd_ref, xbuf, acc, r0)
    plsc.subcore_barrier()

    g_off = c * _E
    s_off = (1 - c) * _E

    def _loop(k, carry):
        e0 = ebase + k * _CH
        pltpu.sync_copy(ei_ref.at[pl.ds(g_off + e0, _CH)], idj)
        pltpu.sync_copy(ei_ref.at[pl.ds(s_off + e0, _CH)], idi)
        pltpu.async_copy(x_ref.at[idj], xbuf, sem).wait()
        pltpu.sync_copy(xbuf, acc.at[idi], add=True)
        return carry
    lax.fori_loop(0, _NCHUNK, _loop, 0)
    plsc.subcore_barrier()

    @pl.when(c == 0)
    def _():
        _drain_acc(acc, xbuf, as_ref, r0)

    @pl.when(c == 1)
    def _():
        _drain_acc(acc, xbuf, at_ref, r0)


def _cnt_body(ei_ref, ones_ref, zn16_ref, cd_ref, cs_ref, cnt, oneb, zbuf, idi):
    c = lax.axis_index("c")
    s = lax.axis_index("s")
    r0 = s * _RPT
    ebase = s * _EPT
    _zero_acc(zn16_ref, zbuf, cnt, r0)
    pltpu.sync_copy(ones_ref, oneb)
    plsc.subcore_barrier()

    key_off = (1 - c) * _E

    def _loop(k, carry):
        e0 = ebase + k * _CH
        pltpu.sync_copy(ei_ref.at[pl.ds(key_off + e0, _CH)], idi)
        pltpu.sync_copy(oneb, cnt.at[idi], add=True)
        return carry
    lax.fori_loop(0, _NCHUNK, _loop, 0)
    plsc.subcore_barrier()

    @pl.when(c == 0)
    def _():
        _drain_acc(cnt, zbuf, cd_ref, r0)

    @pl.when(c == 1)
    def _():
        _drain_acc(cnt, zbuf, cs_ref, r0)


_sc_ea = pl.kernel(
    _ea_body,
    out_type=(
        jax.ShapeDtypeStruct((_NP, _D), jnp.float32),
        jax.ShapeDtypeStruct((_NP, _D), jnp.float32),
    ),
    mesh=_MESH,
    scratch_types=(
        pltpu.VMEM_SHARED((_NP, _D), jnp.float32),
        pltpu.VMEM((_CH, _D), jnp.float32),
        pltpu.VMEM((_CH,), jnp.int32),
    ),
)

_sc_x = pl.kernel(
    _x_body,
    out_type=(
        jax.ShapeDtypeStruct((_NP, _D), jnp.float32),
        jax.ShapeDtypeStruct((_NP, _D), jnp.float32),
    ),
    mesh=_MESH,
    scratch_types=(
        pltpu.VMEM_SHARED((_NP, _D), jnp.float32),
        pltpu.VMEM((_CH, _D), jnp.float32),
        pltpu.VMEM((_CH,), jnp.int32),
        pltpu.VMEM((_CH,), jnp.int32),
        pltpu.SemaphoreType.DMA,
    ),
)

_sc_cnt = pl.kernel(
    _cnt_body,
    out_type=(
        jax.ShapeDtypeStruct((_NP, _D), jnp.float32),
        jax.ShapeDtypeStruct((_NP, _D), jnp.float32),
    ),
    mesh=_MESH,
    scratch_types=(
        pltpu.VMEM_SHARED((_NP, _D), jnp.float32),
        pltpu.VMEM((_CH, _D), jnp.float32),
        pltpu.VMEM((_CH, _D), jnp.float32),
        pltpu.VMEM((_CH,), jnp.int32),
    ),
)


def _tc_combine_body(x_ref, as_ref, at_ref, bd_ref, bs_ref, cd_ref, cs_ref,
                     w_ref, b_ref, o_ref):
    f32 = jnp.float32
    x = x_ref[...]
    o = jnp.dot(as_ref[...], w_ref[0], preferred_element_type=f32)
    o += jnp.dot(at_ref[...], w_ref[1], preferred_element_type=f32)
    o += jnp.dot(bd_ref[...], w_ref[2], preferred_element_type=f32)
    o += jnp.dot(bs_ref[...], w_ref[3], preferred_element_type=f32)
    xqs = jnp.dot(x, w_ref[4], preferred_element_type=f32) + b_ref[0]
    xqt = jnp.dot(x, w_ref[5], preferred_element_type=f32) + b_ref[1]
    o += cd_ref[:, 0:1] * xqs + cs_ref[:, 0:1] * xqt
    o += jnp.dot(x, w_ref[6], preferred_element_type=f32)
    o_ref[...] = o


def _tc_combine(x, As, At, Bd, Bs, cd, cs, Wstack, bstack):
    blk = 2000
    grid = _N // blk
    row = lambda i: (i, 0)
    return pl.pallas_call(
        _tc_combine_body,
        grid=(grid,),
        in_specs=[
            pl.BlockSpec((blk, _D), row),
            pl.BlockSpec((blk, _D), row),
            pl.BlockSpec((blk, _D), row),
            pl.BlockSpec((blk, _D), row),
            pl.BlockSpec((blk, _D), row),
            pl.BlockSpec((blk, _D), row),
            pl.BlockSpec((blk, _D), row),
            pl.BlockSpec((7, _D, _D), lambda i: (0, 0, 0)),
            pl.BlockSpec((2, _D), lambda i: (0, 0)),
        ],
        out_specs=pl.BlockSpec((blk, _D), row),
        out_shape=jax.ShapeDtypeStruct((_N, _D), jnp.float32),
    )(x, As, At, Bd, Bs, cd, cs, Wstack, bstack)


def kernel(x, edge_index, edge_attr, multimodal_features,
           Wn_s2t, We_s2t, Wm_s2t, bm_s2t,
           Wn_t2s, We_t2s, Wm_t2s, bm_t2s):
    D = x.shape[1]
    ei = edge_index.reshape(2 * _E)
    znd = jnp.zeros((_CH, _D), jnp.float32)
    ones128 = jnp.ones((_CH, _D), jnp.float32)

    Bd, Bs = _sc_ea(ei, edge_attr, znd)
    As, At = _sc_x(ei, x, znd)
    cd, cs = _sc_cnt(ei, ones128, znd)

    def mats(Wn, We, Wm):
        W1, W2, W3 = Wm[:, :D], Wm[:, D:2 * D], Wm[:, 2 * D:]
        return Wn.T @ W1.T, Wn.T @ W2.T, We.T @ W3.T

    Ps, Qs, Rs = mats(Wn_s2t, We_s2t, Wm_s2t)
    Pt, Qt, Rt = mats(Wn_t2s, We_t2s, Wm_t2s)
    Wstack = 0.5 * jnp.stack([Ps, Pt, Rs, Rt, Qs, Qt, (Wn_s2t + Wn_t2s).T])
    bstack = 0.5 * jnp.stack([bm_s2t, bm_t2s])

    node_out = _tc_combine(x, As, At, Bd, Bs, cd, cs, Wstack, bstack)
    return (node_out, edge_attr)


# speedup vs baseline: 5.8483x; 1.6670x over previous
import jax
import jax.numpy as jnp
from jax import lax
from jax.experimental import pallas as pl
from jax.experimental.pallas import tpu as pltpu
from jax.experimental.pallas import tpu_sc as plsc

_N = 10000
_E = 320000
_D = 128
_NC = 2
_NS = 16
_CH = 80
_EPT = _E // _NS
_NCHUNK = _EPT // _CH
_NP = 10240
_RPT = _NP // _NS
_QZ = _RPT // _CH

_MESH = plsc.VectorSubcoreMesh(core_axis_name="c", subcore_axis_name="s",
                               num_cores=_NC, num_subcores=_NS)


def _zero_acc(zsrc_ref, buf, acc, r0):
    pltpu.sync_copy(zsrc_ref, buf)

    def _z(q, carry):
        pltpu.sync_copy(buf, acc.at[pl.ds(r0 + q * _CH, _CH)])
        return carry
    lax.fori_loop(0, _QZ, _z, 0)


def _drain_acc(acc, buf, out_ref, r0):
    def _d(q, carry):
        pltpu.sync_copy(acc.at[pl.ds(r0 + q * _CH, _CH)], buf)
        pltpu.sync_copy(buf, out_ref.at[pl.ds(r0 + q * _CH, _CH)])
        return carry
    lax.fori_loop(0, _QZ, _d, 0)


def _ea_body(ei_ref, ea_ref, znd_ref, bd_ref, bs_ref, acc,
             eab0, eab1, id0, id1, se0, se1):
    c = lax.axis_index("c")
    s = lax.axis_index("s")
    r0 = s * _RPT
    ebase = s * _EPT
    _zero_acc(znd_ref, eab0, acc, r0)
    plsc.subcore_barrier()

    key_off = (1 - c) * _E
    bufs = ((eab0, id0, se0), (eab1, id1, se1))

    def _issue(kk, eab, idb, se):
        e0 = ebase + kk * _CH
        pltpu.async_copy(ei_ref.at[pl.ds(key_off + e0, _CH)], idb, se)
        pltpu.async_copy(ea_ref.at[pl.ds(e0, _CH)], eab, se)

    _issue(0, eab0, id0, se0)
    _issue(1, eab1, id1, se1)

    def _loop(k, carry):
        for b in (0, 1):
            eab, idb, se = bufs[b]
            kk = 2 * k + b
            pltpu.make_async_copy(ei_ref.at[pl.ds(0, _CH)], idb, se).wait()
            pltpu.make_async_copy(ea_ref.at[pl.ds(0, _CH)], eab, se).wait()
            pltpu.sync_copy(eab, acc.at[idb], add=True)

            @pl.when(kk + 2 < _NCHUNK)
            def _():
                _issue(kk + 2, eab, idb, se)
        return carry
    lax.fori_loop(0, _NCHUNK // 2, _loop, 0)
    plsc.subcore_barrier()

    @pl.when(c == 0)
    def _():
        _drain_acc(acc, eab0, bd_ref, r0)

    @pl.when(c == 1)
    def _():
        _drain_acc(acc, eab0, bs_ref, r0)


def _x_body(ei_ref, x_ref, znd_ref, as_ref, at_ref, acc,
            xb0, xb1, idi0, idi1, idj0, idj1, si0, si1, sg0, sg1):
    c = lax.axis_index("c")
    s = lax.axis_index("s")
    r0 = s * _RPT
    ebase = s * _EPT
    _zero_acc(znd_ref, xb0, acc, r0)
    plsc.subcore_barrier()

    g_off = c * _E
    s_off = (1 - c) * _E
    bufs = ((xb0, idi0, idj0, si0, sg0), (xb1, idi1, idj1, si1, sg1))

    def _issue_idx(kk, idjb, idib, si):
        e0 = ebase + kk * _CH
        pltpu.async_copy(ei_ref.at[pl.ds(g_off + e0, _CH)], idjb, si)
        pltpu.async_copy(ei_ref.at[pl.ds(s_off + e0, _CH)], idib, si)

    def _wait_idx(idjb, idib, si):
        pltpu.make_async_copy(ei_ref.at[pl.ds(0, _CH)], idjb, si).wait()
        pltpu.make_async_copy(ei_ref.at[pl.ds(0, _CH)], idib, si).wait()

    _issue_idx(0, idj0, idi0, si0)
    _issue_idx(1, idj1, idi1, si1)
    _wait_idx(idj0, idi0, si0)
    pltpu.async_copy(x_ref.at[idj0], xb0, sg0)

    def _loop(k, carry):
        for b in (0, 1):
            xb, idib, idjb, si, sg = bufs[b]
            xbo, idibo, idjbo, sio, sgo = bufs[1 - b]
            kk = 2 * k + b
            pltpu.make_async_copy(x_ref.at[idjb], xb, sg).wait()
            pltpu.sync_copy(xb, acc.at[idib], add=True)

            @pl.when(kk + 2 < _NCHUNK)
            def _():
                _issue_idx(kk + 2, idjb, idib, si)

            @pl.when(kk + 1 < _NCHUNK)
            def _():
                _wait_idx(idjbo, idibo, sio)
                pltpu.async_copy(x_ref.at[idjbo], xbo, sgo)
        return carry
    lax.fori_loop(0, _NCHUNK // 2, _loop, 0)
    plsc.subcore_barrier()

    @pl.when(c == 0)
    def _():
        _drain_acc(acc, xb0, as_ref, r0)

    @pl.when(c == 1)
    def _():
        _drain_acc(acc, xb0, at_ref, r0)


def _cnt_body(ei_ref, ones_ref, zn16_ref, cd_ref, cs_ref, cnt, oneb, zbuf,
              id0, id1, sc0, sc1):
    c = lax.axis_index("c")
    s = lax.axis_index("s")
    r0 = s * _RPT
    ebase = s * _EPT
    _zero_acc(zn16_ref, zbuf, cnt, r0)
    pltpu.sync_copy(ones_ref, oneb)
    plsc.subcore_barrier()

    key_off = (1 - c) * _E
    bufs = ((id0, sc0), (id1, sc1))

    def _issue(kk, idb, sc):
        pltpu.async_copy(ei_ref.at[pl.ds(key_off + ebase + kk * _CH, _CH)], idb, sc)

    _issue(0, id0, sc0)
    _issue(1, id1, sc1)

    def _loop(k, carry):
        for b in (0, 1):
            idb, sc = bufs[b]
            kk = 2 * k + b
            pltpu.make_async_copy(ei_ref.at[pl.ds(0, _CH)], idb, sc).wait()
            pltpu.sync_copy(oneb, cnt.at[idb], add=True)

            @pl.when(kk + 2 < _NCHUNK)
            def _():
                _issue(kk + 2, idb, sc)
        return carry
    lax.fori_loop(0, _NCHUNK // 2, _loop, 0)
    plsc.subcore_barrier()

    @pl.when(c == 0)
    def _():
        _drain_acc(cnt, zbuf, cd_ref, r0)

    @pl.when(c == 1)
    def _():
        _drain_acc(cnt, zbuf, cs_ref, r0)


_sc_ea = pl.kernel(
    _ea_body,
    out_type=(
        jax.ShapeDtypeStruct((_NP, _D), jnp.float32),
        jax.ShapeDtypeStruct((_NP, _D), jnp.float32),
    ),
    mesh=_MESH,
    scratch_types=(
        pltpu.VMEM_SHARED((_NP, _D), jnp.float32),
        pltpu.VMEM((_CH, _D), jnp.float32),
        pltpu.VMEM((_CH, _D), jnp.float32),
        pltpu.VMEM((_CH,), jnp.int32),
        pltpu.VMEM((_CH,), jnp.int32),
        pltpu.SemaphoreType.DMA,
        pltpu.SemaphoreType.DMA,
    ),
)

_sc_x = pl.kernel(
    _x_body,
    out_type=(
        jax.ShapeDtypeStruct((_NP, _D), jnp.float32),
        jax.ShapeDtypeStruct((_NP, _D), jnp.float32),
    ),
    mesh=_MESH,
    scratch_types=(
        pltpu.VMEM_SHARED((_NP, _D), jnp.float32),
        pltpu.VMEM((_CH, _D), jnp.float32),
        pltpu.VMEM((_CH, _D), jnp.float32),
        pltpu.VMEM((_CH,), jnp.int32),
        pltpu.VMEM((_CH,), jnp.int32),
        pltpu.VMEM((_CH,), jnp.int32),
        pltpu.VMEM((_CH,), jnp.int32),
        pltpu.SemaphoreType.DMA,
        pltpu.SemaphoreType.DMA,
        pltpu.SemaphoreType.DMA,
        pltpu.SemaphoreType.DMA,
    ),
)

_sc_cnt = pl.kernel(
    _cnt_body,
    out_type=(
        jax.ShapeDtypeStruct((_NP, _D), jnp.float32),
        jax.ShapeDtypeStruct((_NP, _D), jnp.float32),
    ),
    mesh=_MESH,
    scratch_types=(
        pltpu.VMEM_SHARED((_NP, _D), jnp.float32),
        pltpu.VMEM((_CH, _D), jnp.float32),
        pltpu.VMEM((_CH, _D), jnp.float32),
        pltpu.VMEM((_CH,), jnp.int32),
        pltpu.VMEM((_CH,), jnp.int32),
        pltpu.SemaphoreType.DMA,
        pltpu.SemaphoreType.DMA,
    ),
)


def _tc_combine_body(x_ref, as_ref, at_ref, bd_ref, bs_ref, cd_ref, cs_ref,
                     w_ref, b_ref, o_ref):
    f32 = jnp.float32
    x = x_ref[...]
    o = jnp.dot(as_ref[...], w_ref[0], preferred_element_type=f32)
    o += jnp.dot(at_ref[...], w_ref[1], preferred_element_type=f32)
    o += jnp.dot(bd_ref[...], w_ref[2], preferred_element_type=f32)
    o += jnp.dot(bs_ref[...], w_ref[3], preferred_element_type=f32)
    xqs = jnp.dot(x, w_ref[4], preferred_element_type=f32) + b_ref[0]
    xqt = jnp.dot(x, w_ref[5], preferred_element_type=f32) + b_ref[1]
    o += cd_ref[:, 0:1] * xqs + cs_ref[:, 0:1] * xqt
    o += jnp.dot(x, w_ref[6], preferred_element_type=f32)
    o_ref[...] = o


def _tc_combine(x, As, At, Bd, Bs, cd, cs, Wstack, bstack):
    blk = 2000
    grid = _N // blk
    row = lambda i: (i, 0)
    return pl.pallas_call(
        _tc_combine_body,
        grid=(grid,),
        in_specs=[
            pl.BlockSpec((blk, _D), row),
            pl.BlockSpec((blk, _D), row),
            pl.BlockSpec((blk, _D), row),
            pl.BlockSpec((blk, _D), row),
            pl.BlockSpec((blk, _D), row),
            pl.BlockSpec((blk, _D), row),
            pl.BlockSpec((blk, _D), row),
            pl.BlockSpec((7, _D, _D), lambda i: (0, 0, 0)),
            pl.BlockSpec((2, _D), lambda i: (0, 0)),
        ],
        out_specs=pl.BlockSpec((blk, _D), row),
        out_shape=jax.ShapeDtypeStruct((_N, _D), jnp.float32),
    )(x, As, At, Bd, Bs, cd, cs, Wstack, bstack)


def kernel(x, edge_index, edge_attr, multimodal_features,
           Wn_s2t, We_s2t, Wm_s2t, bm_s2t,
           Wn_t2s, We_t2s, Wm_t2s, bm_t2s):
    D = x.shape[1]
    ei = edge_index.reshape(2 * _E)
    znd = jnp.zeros((_CH, _D), jnp.float32)
    ones128 = jnp.ones((_CH, _D), jnp.float32)

    Bd, Bs = _sc_ea(ei, edge_attr, znd)
    As, At = _sc_x(ei, x, znd)
    cd, cs = _sc_cnt(ei, ones128, znd)

    def mats(Wn, We, Wm):
        W1, W2, W3 = Wm[:, :D], Wm[:, D:2 * D], Wm[:, 2 * D:]
        return Wn.T @ W1.T, Wn.T @ W2.T, We.T @ W3.T

    Ps, Qs, Rs = mats(Wn_s2t, We_s2t, Wm_s2t)
    Pt, Qt, Rt = mats(Wn_t2s, We_t2s, Wm_t2s)
    Wstack = 0.5 * jnp.stack([Ps, Pt, Rs, Rt, Qs, Qt, (Wn_s2t + Wn_t2s).T])
    bstack = 0.5 * jnp.stack([bm_s2t, bm_t2s])

    node_out = _tc_combine(x, As, At, Bd, Bs, cd, cs, Wstack, bstack)
    return (node_out, edge_attr)
